# Initial kernel scaffold; baseline (speedup 1.0000x reference)
import jax
import jax.numpy as jnp
from jax.experimental import pallas as pl


def kernel(bits, group_mem_0, group_mem_1, group_mem_2, group_mem_3, carry_mem_0, carry_mem_1, carry_mem_2):
    raise NotImplementedError("write your pallas kernel here")


# speedup vs baseline: 9.4681x; 9.4681x over previous
import functools

import jax
import jax.numpy as jnp
from jax import lax
from jax.experimental import pallas as pl
from jax.experimental.pallas import tpu as pltpu
from jax.experimental.pallas import tpu_sc as plsc

N_BITS = 32
N_GROUPS = 4
BPG = 8
BATCH = 16384
NC = 2
NS = 16
NW = NC * NS
RPW = BATCH // NW
STEPS = RPW // 16
TBL_ROWS = 256 + 512 + 1024 + 2048
OFF = (0, 256, 768, 1792)
CARRY_LEN = 3 * 256


def _sc_body(bits_hbm, tbl_hbm, carry_hbm, out_hbm, bits_v, tbl_v, carry_v, out_v):
    wid = lax.axis_index("s") * NC + lax.axis_index("c")
    base = pl.multiple_of(wid * (RPW * N_BITS), 8)
    pltpu.sync_copy(bits_hbm.at[pl.ds(base, RPW * N_BITS)], bits_v)
    pltpu.sync_copy(tbl_hbm, tbl_v)
    pltpu.sync_copy(carry_hbm, carry_v)
    lanes = lax.iota(jnp.int32, 16)

    def step(i, acc):
        rb = (i * 16 + lanes) * N_BITS
        addrs = []
        for g in range(N_GROUPS):
            a = plsc.load_gather(bits_v, [rb + g * BPG])
            for j in range(1, BPG):
                b = plsc.load_gather(bits_v, [rb + g * BPG + j])
                a = a + b * (1 << j)
            addrs.append(a)
        cints = []
        for g in range(N_GROUPS - 1):
            c = plsc.load_gather(carry_v, [addrs[g] + 256 * g])
            cints.append(jnp.where(c > 0.5, jnp.int32(1), jnp.int32(0)))
        packs = [
            None,
            cints[0],
            cints[0] + 2 * cints[1],
            cints[0] + 2 * cints[1] + 4 * cints[2],
        ]
        for g in range(N_GROUPS):
            fa = addrs[g] if g == 0 else addrs[g] + 256 * packs[g]
            fi = (fa + OFF[g]) * BPG
            for j in range(BPG):
                v = plsc.load_gather(tbl_v, [fi + j])
                plsc.store_scatter(out_v, [rb + g * BPG + j], v)
        return acc

    lax.fori_loop(0, STEPS, step, 0)
    pltpu.sync_copy(out_v, out_hbm.at[pl.ds(base, RPW * N_BITS)])


@jax.jit
def _mapper(bits_flat, tbl_flat, carry_flat):
    mesh = plsc.VectorSubcoreMesh(core_axis_name="c", subcore_axis_name="s")
    f = pl.kernel(
        _sc_body,
        mesh=mesh,
        compiler_params=pltpu.CompilerParams(needs_layout_passes=False),
        out_type=jax.ShapeDtypeStruct((BATCH * N_BITS,), jnp.float32),
        scratch_types=[
            pltpu.VMEM((RPW * N_BITS,), jnp.int32),
            pltpu.VMEM((TBL_ROWS * BPG,), jnp.float32),
            pltpu.VMEM((CARRY_LEN,), jnp.float32),
            pltpu.VMEM((RPW * N_BITS,), jnp.float32),
        ],
    )
    return f(bits_flat, tbl_flat, carry_flat)


def kernel(bits, group_mem_0, group_mem_1, group_mem_2, group_mem_3,
           carry_mem_0, carry_mem_1, carry_mem_2):
    tbl = jnp.concatenate(
        [group_mem_0.T, group_mem_1.T, group_mem_2.T, group_mem_3.T], axis=0)
    carry = jnp.concatenate([carry_mem_0[0], carry_mem_1[0], carry_mem_2[0]])
    out = _mapper(bits.reshape(-1), tbl.reshape(-1), carry)
    return out.reshape(BATCH, N_BITS)


# speedup vs baseline: 9.5277x; 1.0063x over previous
import functools

import jax
import jax.numpy as jnp
from jax import lax
from jax.experimental import pallas as pl
from jax.experimental.pallas import tpu as pltpu
from jax.experimental.pallas import tpu_sc as plsc

N_BITS = 32
N_GROUPS = 4
BPG = 8
BATCH = 16384
NC = 2
NS = 16
NW = NC * NS
RPW = BATCH // NW
STEPS = RPW // 16


def _sc_body(bits_hbm, t0_hbm, t1_hbm, t2_hbm, t3_hbm, carry_hbm, out_hbm,
             bits_v, t0_v, t1_v, t2_v, t3_v, carry_v, out_v):
    tbls = (t0_v, t1_v, t2_v, t3_v)
    wid = lax.axis_index("s") * NC + lax.axis_index("c")
    base = pl.multiple_of(wid * RPW, 8)
    pltpu.sync_copy(bits_hbm.at[pl.ds(base, RPW)], bits_v)
    pltpu.sync_copy(t0_hbm, t0_v)
    pltpu.sync_copy(t1_hbm, t1_v)
    pltpu.sync_copy(t2_hbm, t2_v)
    pltpu.sync_copy(t3_hbm, t3_v)
    pltpu.sync_copy(carry_hbm, carry_v)
    lanes = lax.iota(jnp.int32, 16)

    def cvec(v):
        return jnp.full((16,), v, jnp.int32)

    def step(i, acc):
        row = i * 16 + lanes
        addrs = []
        for g in range(N_GROUPS):
            a = plsc.load_gather(bits_v, [row, cvec(g * BPG)])
            for j in range(1, BPG):
                b = plsc.load_gather(bits_v, [row, cvec(g * BPG + j)])
                a = a + b * (1 << j)
            addrs.append(a)
        cints = []
        for g in range(N_GROUPS - 1):
            c = plsc.load_gather(carry_v, [addrs[g] + 256 * g])
            cints.append(jnp.where(c > 0.5, jnp.int32(1), jnp.int32(0)))
        packs = [
            None,
            cints[0],
            cints[0] + 2 * cints[1],
            cints[0] + 2 * cints[1] + 4 * cints[2],
        ]
        for g in range(N_GROUPS):
            fa = addrs[g] if g == 0 else addrs[g] + 256 * packs[g]
            for j in range(BPG):
                v = plsc.load_gather(tbls[g], [cvec(j), fa])
                plsc.store_scatter(out_v, [row, cvec(g * BPG + j)], v)
        return acc

    lax.fori_loop(0, STEPS, step, 0)
    pltpu.sync_copy(out_v, out_hbm.at[pl.ds(base, RPW)])


@jax.jit
def _mapper(bits, t0, t1, t2, t3, carry):
    mesh = plsc.VectorSubcoreMesh(core_axis_name="c", subcore_axis_name="s")
    f = pl.kernel(
        _sc_body,
        mesh=mesh,
        compiler_params=pltpu.CompilerParams(
            needs_layout_passes=False, use_tc_tiling_on_sc=False),
        out_type=jax.ShapeDtypeStruct((BATCH, N_BITS), jnp.float32),
        scratch_types=[
            pltpu.VMEM((RPW, N_BITS), jnp.int32),
            pltpu.VMEM((BPG, 256), jnp.float32),
            pltpu.VMEM((BPG, 512), jnp.float32),
            pltpu.VMEM((BPG, 1024), jnp.float32),
            pltpu.VMEM((BPG, 2048), jnp.float32),
            pltpu.VMEM((3 * 256,), jnp.float32),
            pltpu.VMEM((RPW, N_BITS), jnp.float32),
        ],
    )
    return f(bits, t0, t1, t2, t3, carry)


def kernel(bits, group_mem_0, group_mem_1, group_mem_2, group_mem_3,
           carry_mem_0, carry_mem_1, carry_mem_2):
    carry = jnp.concatenate([carry_mem_0[0], carry_mem_1[0], carry_mem_2[0]])
    return _mapper(bits, group_mem_0, group_mem_1, group_mem_2, group_mem_3,
                   carry)


# speedup vs baseline: 9.7532x; 1.0237x over previous
import functools

import jax
import jax.numpy as jnp
from jax import lax
from jax.experimental import pallas as pl
from jax.experimental.pallas import tpu as pltpu
from jax.experimental.pallas import tpu_sc as plsc

N_BITS = 32
N_GROUPS = 4
BPG = 8
BATCH = 16384
NC = 2
NS = 16
NW = NC * NS
RPW = BATCH // NW
STEPS = RPW // 16


def _sc_body(bits_hbm, t0_hbm, t1_hbm, t2_hbm, t3_hbm, carry_hbm, out_hbm,
             bits_v, t0_v, t1_v, t2_v, t3_v, carry_v, out_v):
    tbls = (t0_v, t1_v, t2_v, t3_v)
    wid = lax.axis_index("s") * NC + lax.axis_index("c")
    base = pl.multiple_of(wid * RPW, 8)
    pltpu.sync_copy(bits_hbm.at[pl.ds(base, RPW)], bits_v)
    pltpu.sync_copy(t0_hbm, t0_v)
    pltpu.sync_copy(t1_hbm, t1_v)
    pltpu.sync_copy(t2_hbm, t2_v)
    pltpu.sync_copy(t3_hbm, t3_v)
    pltpu.sync_copy(carry_hbm, carry_v)
    lanes = lax.iota(jnp.int32, 16)

    def cvec(v):
        return jnp.full((16,), v, jnp.int32)

    @plsc.parallel_loop(0, STEPS, unroll=4)
    def step(i):
        row = i * 16 + lanes
        addrs = []
        for g in range(N_GROUPS):
            b = [plsc.load_gather(bits_v, [row, cvec(g * BPG + j)])
                 for j in range(BPG)]
            a = ((b[0] + 2 * b[1]) + 4 * (b[2] + 2 * b[3])) + 16 * (
                (b[4] + 2 * b[5]) + 4 * (b[6] + 2 * b[7]))
            addrs.append(a)
        cints = []
        for g in range(N_GROUPS - 1):
            c = plsc.load_gather(carry_v, [addrs[g] + 256 * g])
            cints.append(jnp.where(c > 0.5, jnp.int32(1), jnp.int32(0)))
        packs = [
            None,
            cints[0],
            cints[0] + 2 * cints[1],
            cints[0] + 2 * cints[1] + 4 * cints[2],
        ]
        for g in range(N_GROUPS):
            fa = addrs[g] if g == 0 else addrs[g] + 256 * packs[g]
            for j in range(BPG):
                v = plsc.load_gather(tbls[g], [cvec(j), fa])
                plsc.store_scatter(out_v, [row, cvec(g * BPG + j)], v)

    pltpu.sync_copy(out_v, out_hbm.at[pl.ds(base, RPW)])


@jax.jit
def _mapper(bits, t0, t1, t2, t3, carry):
    mesh = plsc.VectorSubcoreMesh(core_axis_name="c", subcore_axis_name="s")
    f = pl.kernel(
        _sc_body,
        mesh=mesh,
        compiler_params=pltpu.CompilerParams(
            needs_layout_passes=False, use_tc_tiling_on_sc=False),
        out_type=jax.ShapeDtypeStruct((BATCH, N_BITS), jnp.float32),
        scratch_types=[
            pltpu.VMEM((RPW, N_BITS), jnp.int32),
            pltpu.VMEM((BPG, 256), jnp.float32),
            pltpu.VMEM((BPG, 512), jnp.float32),
            pltpu.VMEM((BPG, 1024), jnp.float32),
            pltpu.VMEM((BPG, 2048), jnp.float32),
            pltpu.VMEM((3 * 256,), jnp.float32),
            pltpu.VMEM((RPW, N_BITS), jnp.float32),
        ],
    )
    return f(bits, t0, t1, t2, t3, carry)


def kernel(bits, group_mem_0, group_mem_1, group_mem_2, group_mem_3,
           carry_mem_0, carry_mem_1, carry_mem_2):
    carry = jnp.concatenate([carry_mem_0[0], carry_mem_1[0], carry_mem_2[0]])
    return _mapper(bits, group_mem_0, group_mem_1, group_mem_2, group_mem_3,
                   carry)


# speedup vs baseline: 16.2382x; 1.6649x over previous
import functools

import jax
import jax.numpy as jnp
from jax import lax
from jax.experimental import pallas as pl
from jax.experimental.pallas import tpu as pltpu
from jax.experimental.pallas import tpu_sc as plsc

N_BITS = 32
N_GROUPS = 4
BPG = 8
BATCH = 16384
NC = 2
NS = 16
NW = NC * NS
RPW = BATCH // NW
STEPS = RPW // 16


def _sc_body(bits_hbm, t0_hbm, t1_hbm, t2_hbm, t3_hbm, carry_hbm, out_hbm,
             bits_v, t0_v, t1_v, t2_v, t3_v, carry_v, out_v):
    tbls = (t0_v, t1_v, t2_v, t3_v)
    wid = lax.axis_index("s") * NC + lax.axis_index("c")
    base = pl.multiple_of(wid * RPW, 8)
    pltpu.sync_copy(bits_hbm.at[:, pl.ds(base, RPW)], bits_v)
    pltpu.sync_copy(t0_hbm, t0_v)
    pltpu.sync_copy(t1_hbm, t1_v)
    pltpu.sync_copy(t2_hbm, t2_v)
    pltpu.sync_copy(t3_hbm, t3_v)
    pltpu.sync_copy(carry_hbm, carry_v)

    def cvec(v):
        return jnp.full((16,), v, jnp.int32)

    @plsc.parallel_loop(0, STEPS, unroll=2)
    def step(i):
        r = i * 16
        addrs = []
        for g in range(N_GROUPS):
            b = [bits_v[g * BPG + j, pl.ds(r, 16)] for j in range(BPG)]
            a = ((b[0] + 2 * b[1]) + 4 * (b[2] + 2 * b[3])) + 16 * (
                (b[4] + 2 * b[5]) + 4 * (b[6] + 2 * b[7]))
            addrs.append(a)
        cints = []
        for g in range(N_GROUPS - 1):
            c = plsc.load_gather(carry_v, [addrs[g] + 256 * g])
            cints.append(jnp.where(c > 0.5, jnp.int32(1), jnp.int32(0)))
        packs = [
            None,
            cints[0],
            cints[0] + 2 * cints[1],
            cints[0] + 2 * cints[1] + 4 * cints[2],
        ]
        for g in range(N_GROUPS):
            fa = addrs[g] if g == 0 else addrs[g] + 256 * packs[g]
            for j in range(BPG):
                out_v[g * BPG + j, pl.ds(r, 16)] = (
                    plsc.load_gather(tbls[g], [cvec(j), fa]))

    pltpu.sync_copy(out_v, out_hbm.at[:, pl.ds(base, RPW)])


@jax.jit
def _mapper(bits_t, t0, t1, t2, t3, carry):
    mesh = plsc.VectorSubcoreMesh(core_axis_name="c", subcore_axis_name="s")
    f = pl.kernel(
        _sc_body,
        mesh=mesh,
        compiler_params=pltpu.CompilerParams(
            needs_layout_passes=False, use_tc_tiling_on_sc=False),
        out_type=jax.ShapeDtypeStruct((N_BITS, BATCH), jnp.float32),
        scratch_types=[
            pltpu.VMEM((N_BITS, RPW), jnp.int32),
            pltpu.VMEM((BPG, 256), jnp.float32),
            pltpu.VMEM((BPG, 512), jnp.float32),
            pltpu.VMEM((BPG, 1024), jnp.float32),
            pltpu.VMEM((BPG, 2048), jnp.float32),
            pltpu.VMEM((3 * 256,), jnp.float32),
            pltpu.VMEM((N_BITS, RPW), jnp.float32),
        ],
    )
    return f(bits_t, t0, t1, t2, t3, carry)


def kernel(bits, group_mem_0, group_mem_1, group_mem_2, group_mem_3,
           carry_mem_0, carry_mem_1, carry_mem_2):
    carry = jnp.concatenate([carry_mem_0[0], carry_mem_1[0], carry_mem_2[0]])
    out_t = _mapper(bits.T, group_mem_0, group_mem_1, group_mem_2,
                    group_mem_3, carry)
    return out_t.T


# speedup vs baseline: 18.7462x; 1.1545x over previous
import functools

import jax
import jax.numpy as jnp
from jax import lax
from jax.experimental import pallas as pl
from jax.experimental.pallas import tpu as pltpu
from jax.experimental.pallas import tpu_sc as plsc

N_BITS = 32
N_GROUPS = 4
BPG = 8
BATCH = 16384
NC = 2
NS = 16
NW = NC * NS
RPW = BATCH // NW
STEPS = RPW // 16


TOFF = (0, 256, 768, 1792)
TCOLS = 3840


def _sc_body(bits_hbm, tbl_hbm, carry_hbm, out_hbm,
             bits_v, tbl_v, carry_v, out_v):
    wid = lax.axis_index("s") * NC + lax.axis_index("c")
    base = pl.multiple_of(wid * RPW, 8)
    pltpu.sync_copy(bits_hbm.at[:, pl.ds(base, RPW)], bits_v)
    pltpu.sync_copy(tbl_hbm, tbl_v)
    pltpu.sync_copy(carry_hbm, carry_v)

    def cvec(v):
        return jnp.full((16,), v, jnp.int32)

    @plsc.parallel_loop(0, STEPS, unroll=2)
    def step(i):
        r = i * 16
        addrs = []
        for g in range(N_GROUPS):
            b = [bits_v[g * BPG + j, pl.ds(r, 16)] for j in range(BPG)]
            a = ((b[0] + 2 * b[1]) + 4 * (b[2] + 2 * b[3])) + 16 * (
                (b[4] + 2 * b[5]) + 4 * (b[6] + 2 * b[7]))
            addrs.append(a)
        cints = []
        for g in range(N_GROUPS - 1):
            c = plsc.load_gather(carry_v, [addrs[g] + 256 * g])
            cints.append(jnp.where(c > 0.5, jnp.int32(1), jnp.int32(0)))
        packs = [
            None,
            cints[0],
            cints[0] + 2 * cints[1],
            cints[0] + 2 * cints[1] + 4 * cints[2],
        ]
        for g in range(N_GROUPS):
            fa = addrs[g] if g == 0 else addrs[g] + 256 * packs[g]
            fa = fa + TOFF[g]
            for j in range(BPG):
                out_v[g * BPG + j, pl.ds(r, 16)] = (
                    plsc.load_gather(tbl_v, [cvec(j), fa]))

    pltpu.sync_copy(out_v, out_hbm.at[:, pl.ds(base, RPW)])


@jax.jit
def _mapper(bits_t, tbl, carry):
    mesh = plsc.VectorSubcoreMesh(core_axis_name="c", subcore_axis_name="s")
    f = pl.kernel(
        _sc_body,
        mesh=mesh,
        compiler_params=pltpu.CompilerParams(
            needs_layout_passes=False, use_tc_tiling_on_sc=False),
        out_type=jax.ShapeDtypeStruct((N_BITS, BATCH), jnp.float32),
        scratch_types=[
            pltpu.VMEM((N_BITS, RPW), jnp.int32),
            pltpu.VMEM((BPG, TCOLS), jnp.float32),
            pltpu.VMEM((3 * 256,), jnp.float32),
            pltpu.VMEM((N_BITS, RPW), jnp.float32),
        ],
    )
    return f(bits_t, tbl, carry)


def kernel(bits, group_mem_0, group_mem_1, group_mem_2, group_mem_3,
           carry_mem_0, carry_mem_1, carry_mem_2):
    tbl = jnp.concatenate(
        [group_mem_0, group_mem_1, group_mem_2, group_mem_3], axis=1)
    carry = jnp.concatenate([carry_mem_0[0], carry_mem_1[0], carry_mem_2[0]])
    out_t = _mapper(bits.T, tbl, carry)
    return out_t.T


# speedup vs baseline: 20.1244x; 1.0735x over previous
import functools

import jax
import jax.numpy as jnp
from jax import lax
from jax.experimental import pallas as pl
from jax.experimental.pallas import tpu as pltpu
from jax.experimental.pallas import tpu_sc as plsc

N_BITS = 32
N_GROUPS = 4
BPG = 8
BATCH = 16384
NC = 2
NS = 16
NW = NC * NS
RPW = BATCH // NW
STEPS = RPW // 16
NPLANES = 8
MAGIC = 0x01020408
TOFF = (0, 256, 768, 1792)
TCOLS = 3840


def _sc_body(bits_hbm, tbl_hbm, carry_hbm, out_hbm,
             bits_v, tbl_v, carry_v, out_v, sem):
    wid = lax.axis_index("s") * NC + lax.axis_index("c")
    base = pl.multiple_of(wid * RPW, 8)
    c1 = pltpu.async_copy(bits_hbm.at[:, pl.ds(base, RPW)], bits_v, sem)
    c2 = pltpu.async_copy(tbl_hbm, tbl_v, sem)
    c3 = pltpu.async_copy(carry_hbm, carry_v, sem)
    c1.wait()
    c2.wait()
    c3.wait()

    def cvec(v):
        return jnp.full((16,), v, jnp.int32)

    @plsc.parallel_loop(0, STEPS, unroll=2)
    def step(i):
        r = i * 16
        addrs = []
        for g in range(N_GROUPS):
            w_lo = bits_v[2 * g, pl.ds(r, 16)]
            w_hi = bits_v[2 * g + 1, pl.ds(r, 16)]
            addrs.append(((w_lo * MAGIC) >> 24)
                         + 16 * ((w_hi * MAGIC) >> 24))
        cints = []
        for g in range(N_GROUPS - 1):
            c = plsc.load_gather(carry_v, [addrs[g] + 256 * g])
            cints.append(jnp.where(c > 0.5, jnp.int32(1), jnp.int32(0)))
        packs = [
            None,
            cints[0],
            cints[0] + 2 * cints[1],
            cints[0] + 2 * cints[1] + 4 * cints[2],
        ]
        for g in range(N_GROUPS):
            fa = addrs[g] if g == 0 else addrs[g] + 256 * packs[g]
            fa = fa + TOFF[g]
            for j in range(BPG):
                out_v[g * BPG + j, pl.ds(r, 16)] = (
                    plsc.load_gather(tbl_v, [cvec(j), fa]))

    pltpu.sync_copy(out_v, out_hbm.at[:, pl.ds(base, RPW)])


@jax.jit
def _mapper(pbits, tbl, carry):
    mesh = plsc.VectorSubcoreMesh(core_axis_name="c", subcore_axis_name="s")
    f = pl.kernel(
        _sc_body,
        mesh=mesh,
        compiler_params=pltpu.CompilerParams(
            needs_layout_passes=False, use_tc_tiling_on_sc=False),
        out_type=jax.ShapeDtypeStruct((N_BITS, BATCH), jnp.float32),
        scratch_types=[
            pltpu.VMEM((NPLANES, RPW), jnp.int32),
            pltpu.VMEM((BPG, TCOLS), jnp.float32),
            pltpu.VMEM((3 * 256,), jnp.float32),
            pltpu.VMEM((N_BITS, RPW), jnp.float32),
            pltpu.SemaphoreType.DMA,
        ],
    )
    return f(pbits, tbl, carry)


def kernel(bits, group_mem_0, group_mem_1, group_mem_2, group_mem_3,
           carry_mem_0, carry_mem_1, carry_mem_2):
    pbits = jax.lax.bitcast_convert_type(
        bits.astype(jnp.int8).reshape(BATCH, NPLANES, 4), jnp.int32).T
    tbl = jnp.concatenate(
        [group_mem_0, group_mem_1, group_mem_2, group_mem_3], axis=1)
    carry = jnp.concatenate([carry_mem_0[0], carry_mem_1[0], carry_mem_2[0]])
    out_t = _mapper(pbits, tbl, carry)
    return out_t.T


# speedup vs baseline: 20.6295x; 1.0251x over previous
import functools

import jax
import jax.numpy as jnp
from jax import lax
from jax.experimental import pallas as pl
from jax.experimental.pallas import tpu as pltpu
from jax.experimental.pallas import tpu_sc as plsc

N_BITS = 32
N_GROUPS = 4
BPG = 8
BATCH = 16384
NC = 2
NS = 16
NW = NC * NS
RPW = BATCH // NW
STEPS = RPW // 16
NPLANES = 8
MAGIC = 0x01020408
TOFF = (0, 256, 768, 1792)
TCOLS = 3840


def _sc_body(bits_hbm, tbl_hbm, carry_hbm, out_hbm,
             bits_v, tbl_v, carry_v, out_v, sem):
    wid = lax.axis_index("s") * NC + lax.axis_index("c")
    base = pl.multiple_of(wid * RPW, 8)
    c1 = pltpu.async_copy(bits_hbm.at[:, pl.ds(base, RPW)], bits_v, sem)
    c2 = pltpu.async_copy(tbl_hbm, tbl_v, sem)
    c3 = pltpu.async_copy(carry_hbm, carry_v, sem)
    c1.wait()
    c2.wait()
    c3.wait()

    def cvec(v):
        return jnp.full((16,), v, jnp.int32)

    @plsc.parallel_loop(0, STEPS, unroll=1)
    def step(i):
        r = i * 16
        addrs = []
        for g in range(N_GROUPS):
            w_lo = bits_v[2 * g, pl.ds(r, 16)]
            w_hi = bits_v[2 * g + 1, pl.ds(r, 16)]
            addrs.append(((w_lo * MAGIC) >> 24)
                         + 16 * ((w_hi * MAGIC) >> 24))
        cints = []
        for g in range(N_GROUPS - 1):
            c = plsc.load_gather(carry_v, [addrs[g] + 256 * g])
            cints.append(jnp.where(c > 0.5, jnp.int32(1), jnp.int32(0)))
        packs = [
            None,
            cints[0],
            cints[0] + 2 * cints[1],
            cints[0] + 2 * cints[1] + 4 * cints[2],
        ]
        for g in range(N_GROUPS):
            fa = addrs[g] if g == 0 else addrs[g] + 256 * packs[g]
            fa = fa + TOFF[g]
            for j in range(BPG):
                out_v[g * BPG + j, pl.ds(r, 16)] = (
                    plsc.load_gather(tbl_v, [cvec(j), fa]))

    pltpu.sync_copy(out_v, out_hbm.at[:, pl.ds(base, RPW)])


@jax.jit
def _mapper(pbits, tbl, carry):
    mesh = plsc.VectorSubcoreMesh(core_axis_name="c", subcore_axis_name="s")
    f = pl.kernel(
        _sc_body,
        mesh=mesh,
        compiler_params=pltpu.CompilerParams(
            needs_layout_passes=False, use_tc_tiling_on_sc=False),
        out_type=jax.ShapeDtypeStruct((N_BITS, BATCH), jnp.float32),
        scratch_types=[
            pltpu.VMEM((NPLANES, RPW), jnp.int32),
            pltpu.VMEM((BPG, TCOLS), jnp.float32),
            pltpu.VMEM((3 * 256,), jnp.float32),
            pltpu.VMEM((N_BITS, RPW), jnp.float32),
            pltpu.SemaphoreType.DMA,
        ],
    )
    return f(pbits, tbl, carry)


def kernel(bits, group_mem_0, group_mem_1, group_mem_2, group_mem_3,
           carry_mem_0, carry_mem_1, carry_mem_2):
    pbits = jax.lax.bitcast_convert_type(
        bits.astype(jnp.int8).reshape(BATCH, NPLANES, 4), jnp.int32).T
    tbl = jnp.concatenate(
        [group_mem_0, group_mem_1, group_mem_2, group_mem_3], axis=1)
    carry = jnp.concatenate([carry_mem_0[0], carry_mem_1[0], carry_mem_2[0]])
    out_t = _mapper(pbits, tbl, carry)
    return out_t.T


# speedup vs baseline: 23.5571x; 1.1419x over previous
import functools

import jax
import jax.numpy as jnp
from jax import lax
from jax.experimental import pallas as pl
from jax.experimental.pallas import tpu as pltpu
from jax.experimental.pallas import tpu_sc as plsc

N_BITS = 32
N_GROUPS = 4
BPG = 8
BATCH = 16384
NC = 2
NS = 16
NW = NC * NS
RPW = BATCH // NW
STEPS = RPW // 16
NPLANES = 8
MAGIC = 0x01020408
TOFF = (0, 256, 768, 1792)
TCOLS = 3840


def _sc_body(bits_hbm, tbl_hbm, carry_hbm, out_hbm,
             bits_v, tbl_v, carry_v, out_v, sem):
    wid = lax.axis_index("s") * NC + lax.axis_index("c")
    base = pl.multiple_of(wid * RPW, 8)
    c1 = pltpu.async_copy(bits_hbm.at[:, pl.ds(base, RPW)], bits_v, sem)
    c2 = pltpu.async_copy(tbl_hbm, tbl_v, sem)
    c3 = pltpu.async_copy(carry_hbm, carry_v, sem)
    c1.wait()
    c2.wait()
    c3.wait()

    def cvec(v):
        return jnp.full((16,), v, jnp.int32)

    @plsc.parallel_loop(0, STEPS, unroll=1)
    def step(i):
        r = i * 16
        addrs = []
        for g in range(N_GROUPS):
            w_lo = bits_v[2 * g, pl.ds(r, 16)]
            w_hi = bits_v[2 * g + 1, pl.ds(r, 16)]
            addrs.append(((w_lo * MAGIC) >> 24)
                         + 16 * ((w_hi * MAGIC) >> 24))
        cints = []
        for g in range(N_GROUPS - 1):
            c = plsc.load_gather(carry_v, [addrs[g] + 256 * g])
            cints.append(jnp.where(c > 0.5, jnp.int32(1), jnp.int32(0)))
        packs = [
            None,
            cints[0],
            cints[0] + 2 * cints[1],
            cints[0] + 2 * cints[1] + 4 * cints[2],
        ]
        for g in range(N_GROUPS):
            fa = addrs[g] if g == 0 else addrs[g] + 256 * packs[g]
            w = plsc.load_gather(tbl_v, [fa + TOFF[g]])
            for j in range(BPG):
                out_v[g * BPG + j, pl.ds(r, 16)] = (
                    ((w >> j) & 1).astype(jnp.float32))

    pltpu.sync_copy(out_v, out_hbm.at[:, pl.ds(base, RPW)])


@jax.jit
def _mapper(pbits, tbl, carry):
    mesh = plsc.VectorSubcoreMesh(core_axis_name="c", subcore_axis_name="s")
    f = pl.kernel(
        _sc_body,
        mesh=mesh,
        compiler_params=pltpu.CompilerParams(
            needs_layout_passes=False, use_tc_tiling_on_sc=False),
        out_type=jax.ShapeDtypeStruct((N_BITS, BATCH), jnp.float32),
        scratch_types=[
            pltpu.VMEM((NPLANES, RPW), jnp.int32),
            pltpu.VMEM((TCOLS,), jnp.int32),
            pltpu.VMEM((3 * 256,), jnp.float32),
            pltpu.VMEM((N_BITS, RPW), jnp.float32),
            pltpu.SemaphoreType.DMA,
        ],
    )
    return f(pbits, tbl, carry)


def kernel(bits, group_mem_0, group_mem_1, group_mem_2, group_mem_3,
           carry_mem_0, carry_mem_1, carry_mem_2):
    pbits = jax.lax.bitcast_convert_type(
        bits.astype(jnp.int8).reshape(BATCH, NPLANES, 4), jnp.int32).T
    tblf = jnp.concatenate(
        [group_mem_0, group_mem_1, group_mem_2, group_mem_3], axis=1)
    tbl = jnp.sum(
        tblf.astype(jnp.int32) << jnp.arange(BPG, dtype=jnp.int32)[:, None],
        axis=0, dtype=jnp.int32)
    carry = jnp.concatenate([carry_mem_0[0], carry_mem_1[0], carry_mem_2[0]])
    out_t = _mapper(pbits, tbl, carry)
    return out_t.T
